# baseline (speedup 1.0000x reference)
import functools

import jax
import jax.numpy as jnp
from jax import lax
from jax.experimental import pallas as pl
from jax.experimental.pallas import tpu as pltpu
from jax.experimental.pallas import tpu_sc as plsc

N = 10000
E = 320000
D = 128
NC = 2
NS = 16
NW = NC * NS
CHUNK = 128
CPW = 80
EPW = CPW * CHUNK
E_PAD = EPW * NW
N_ACC = 10240
RPS = N_ACC // NS
DUMMY = N + 8


SB = 8
NSB = CPW // SB


def _sc_agg_body(compute_deg, g_hbm, h_hbm, z_hbm, *rest):
    if compute_deg:
        (agg_out, deg_out, idx4, buf_a, buf_b, deg_v, acc_sh,
         sem_a, sem_b, sem_i) = rest
    else:
        agg_out, idx4, buf_a, buf_b, acc_sh, sem_a, sem_b, sem_i = rest
        deg_out = deg_v = None

    cid = lax.axis_index("c")
    sid = lax.axis_index("s")
    w = cid * NS + sid
    bufs = (buf_a, buf_b)
    sems = (sem_a, sem_b)

    pltpu.sync_copy(z_hbm, buf_a)
    for r in range(RPS // CHUNK):
        pltpu.sync_copy(buf_a, acc_sh.at[pl.ds(sid * RPS + r * CHUNK, CHUNK)])

    if compute_deg:
        z16 = jnp.zeros((16,), jnp.float32)

        def zbody(i, carry):
            deg_v[pl.ds(i * 16, 16)] = z16
            return carry

        lax.fori_loop(0, N_ACC // 16, zbody, 0)

    plsc.subcore_barrier()

    ones16 = jnp.ones((16,), jnp.float32)

    def deg_update(p, j):
        if compute_deg:
            for k in range(CHUNK // 16):
                idx16 = idx4[p, j, 1, pl.ds(k * 16, 16)]
                plsc.addupdate_scatter(deg_v, [idx16], ones16)

    pltpu.sync_copy(g_hbm.at[w, 0], idx4.at[0])
    pltpu.async_copy(g_hbm.at[w, 1], idx4.at[1], sem_i)
    pltpu.async_copy(h_hbm.at[idx4.at[0, 0, 0]], buf_a, sem_a)

    def sb_body(s, carry):
        p = lax.rem(s, 2)
        pn = lax.rem(s + 1, 2)
        for j in range(SB):
            if j == SB - 2:
                pltpu.make_async_copy(
                    g_hbm.at[w, jnp.minimum(s + 1, NSB - 1)],
                    idx4.at[pn], sem_i).wait()
            pltpu.make_async_copy(
                h_hbm.at[idx4.at[p, j, 0]], bufs[j % 2], sems[j % 2]).wait()
            if j < SB - 1:
                nxt_idx = idx4.at[p, j + 1, 0]
            else:
                nxt_idx = idx4.at[pn, 0, 0]
            pltpu.async_copy(h_hbm.at[nxt_idx], bufs[(j + 1) % 2],
                             sems[(j + 1) % 2])
            pltpu.sync_copy(bufs[j % 2], acc_sh.at[idx4.at[p, j, 1]],
                            add=True)
            deg_update(p, j)
        pltpu.async_copy(g_hbm.at[w, jnp.minimum(s + 2, NSB - 1)],
                         idx4.at[p], sem_i)
        return carry

    lax.fori_loop(0, NSB, sb_body, 0)
    pltpu.make_async_copy(g_hbm.at[w, NSB - 1],
                          idx4.at[(NSB - 1) % 2], sem_i).wait()
    pltpu.make_async_copy(h_hbm.at[idx4.at[0, 0, 0]], buf_a, sem_a).wait()

    plsc.subcore_barrier()

    pltpu.sync_copy(acc_sh.at[pl.ds(sid * RPS, RPS)],
                    agg_out.at[cid, pl.ds(sid * RPS, RPS)])
    if compute_deg:
        pltpu.sync_copy(deg_v, deg_out.at[w])


def _make_sc_agg(compute_deg):
    out_type = [jax.ShapeDtypeStruct((NC, N_ACC, D), jnp.float32)]
    scratch = [
        pltpu.VMEM((2, SB, 2, CHUNK), jnp.int32),
        pltpu.VMEM((CHUNK, D), jnp.float32),
        pltpu.VMEM((CHUNK, D), jnp.float32),
    ]
    if compute_deg:
        out_type.append(jax.ShapeDtypeStruct((NW, N_ACC), jnp.float32))
        scratch.append(pltpu.VMEM((N_ACC,), jnp.float32))
    scratch.append(pltpu.VMEM_SHARED((N_ACC, D), jnp.float32))
    scratch.append(pltpu.SemaphoreType.DMA)
    scratch.append(pltpu.SemaphoreType.DMA)
    scratch.append(pltpu.SemaphoreType.DMA)

    return pl.kernel(
        functools.partial(_sc_agg_body, compute_deg),
        out_type=out_type,
        mesh=plsc.VectorSubcoreMesh(core_axis_name="c", subcore_axis_name="s"),
        scratch_types=scratch,
        compiler_params=pltpu.CompilerParams(needs_layout_passes=False),
    )


_sc_agg_deg = _make_sc_agg(True)
_sc_agg = _make_sc_agg(False)


_R = 400


def _self_body(h_ref, ws_ref, b_ref, o_ref):
    o_ref[...] = jnp.dot(h_ref[...], ws_ref[...],
                         preferred_element_type=jnp.float32) + b_ref[0:1, :]


def _dense_self(h, W_self, b2d):
    return pl.pallas_call(
        _self_body,
        grid=(N // _R,),
        in_specs=[
            pl.BlockSpec((_R, D), lambda i: (i, 0)),
            pl.BlockSpec((D, D), lambda i: (0, 0)),
            pl.BlockSpec((8, D), lambda i: (0, 0)),
        ],
        out_specs=pl.BlockSpec((_R, D), lambda i: (i, 0)),
        out_shape=jax.ShapeDtypeStruct((N, D), jnp.float32),
    )(h, W_self, b2d)


def _combine_body(s_ref, a_ref, d_ref, wn_ref, o_ref):
    a = a_ref[0] + a_ref[1]
    deg = jnp.sum(d_ref[...], axis=1)
    inv = 1.0 / jnp.maximum(deg, 1.0)
    hn = a * inv[:, None]
    acc = s_ref[...] + jnp.dot(hn, wn_ref[...],
                               preferred_element_type=jnp.float32)
    o_ref[...] = jnp.maximum(acc, 0.0)


def _dense_combine(self_part, agg, deg_parts, W_neigh):
    return pl.pallas_call(
        _combine_body,
        grid=(N // _R,),
        in_specs=[
            pl.BlockSpec((_R, D), lambda i: (i, 0)),
            pl.BlockSpec((NC, _R, D), lambda i: (0, i, 0)),
            pl.BlockSpec((_R, NW), lambda i: (i, 0)),
            pl.BlockSpec((D, D), lambda i: (0, 0)),
        ],
        out_specs=pl.BlockSpec((_R, D), lambda i: (i, 0)),
        out_shape=jax.ShapeDtypeStruct((N, D), jnp.float32),
    )(self_part, agg, deg_parts, W_neigh)


@jax.jit
def kernel(g, inputs, W_self1, W_neigh1, b1, W_self2, W_neigh2, b2):
    pad = E_PAD - E
    pad_src = (jnp.arange(pad, dtype=jnp.int32) * 131) % N
    pad_dst = DUMMY + (jnp.arange(pad, dtype=jnp.int32) % 32)
    src_p = jnp.concatenate([g[0], pad_src]).reshape(NW, CPW, CHUNK)
    dst_p = jnp.concatenate([g[1], pad_dst]).reshape(NW, CPW, CHUNK)
    g_p = jnp.stack([src_p, dst_p], axis=2).reshape(
        NW, NSB, SB, 2, CHUNK)
    z = jnp.zeros((CHUNK, D), jnp.float32)
    b1_2 = jnp.broadcast_to(b1, (8, D))
    b2_2 = jnp.broadcast_to(b2, (8, D))

    agg1, deg_parts = _sc_agg_deg(g_p, inputs, z)
    deg_t = deg_parts.T
    self1 = _dense_self(inputs, W_self1, b1_2)
    h1 = _dense_combine(self1, agg1, deg_t, W_neigh1)
    (agg2,) = _sc_agg(g_p, h1, z)
    self2 = _dense_self(h1, W_self2, b2_2)
    h2 = _dense_combine(self2, agg2, deg_t, W_neigh2)
    return h2


# speedup vs baseline: 1.0296x; 1.0296x over previous
import functools

import jax
import jax.numpy as jnp
from jax import lax
from jax.experimental import pallas as pl
from jax.experimental.pallas import tpu as pltpu
from jax.experimental.pallas import tpu_sc as plsc

N = 10000
E = 320000
D = 128
NC = 2
NS = 16
NW = NC * NS
CHUNK = 128
CPW = 80
EPW = CPW * CHUNK
E_PAD = EPW * NW
N_ACC = 10240
RPS = N_ACC // NS
DUMMY = N + 8


SB = 8
NSB = CPW // SB


def _sc_agg_body(compute_deg, g_hbm, h_hbm, z_hbm, *rest):
    if compute_deg:
        (agg_out, deg_out, idx_s, idx_d, buf_a, buf_b, deg_v, acc_sh,
         sem_a, sem_b, sem_i) = rest
    else:
        agg_out, idx_s, idx_d, buf_a, buf_b, acc_sh, sem_a, sem_b, sem_i = rest
        deg_out = deg_v = None

    cid = lax.axis_index("c")
    sid = lax.axis_index("s")
    w = cid * NS + sid
    bufs = (buf_a, buf_b)
    sems = (sem_a, sem_b)

    pltpu.sync_copy(z_hbm, buf_a)
    for r in range(RPS // CHUNK):
        pltpu.sync_copy(buf_a, acc_sh.at[pl.ds(sid * RPS + r * CHUNK, CHUNK)])

    if compute_deg:
        z16 = jnp.zeros((16,), jnp.float32)

        def zbody(i, carry):
            deg_v[pl.ds(i * 16, 16)] = z16
            return carry

        lax.fori_loop(0, N_ACC // 16, zbody, 0)

    plsc.subcore_barrier()

    ones16 = jnp.ones((16,), jnp.float32)

    def deg_update(p, j):
        if compute_deg:
            for k in range(CHUNK // 16):
                idx16 = idx_d[p, j, pl.ds(k * 16, 16)]
                plsc.addupdate_scatter(deg_v, [idx16], ones16)

    def idx_load(s_clamped, half, sem):
        pltpu.async_copy(g_hbm.at[0, w, s_clamped], idx_s.at[half], sem)
        pltpu.async_copy(g_hbm.at[1, w, s_clamped], idx_d.at[half], sem)

    def idx_wait(s_clamped, half, sem):
        pltpu.make_async_copy(g_hbm.at[0, w, s_clamped], idx_s.at[half],
                              sem).wait()
        pltpu.make_async_copy(g_hbm.at[1, w, s_clamped], idx_d.at[half],
                              sem).wait()

    pltpu.sync_copy(g_hbm.at[0, w, 0], idx_s.at[0])
    pltpu.sync_copy(g_hbm.at[1, w, 0], idx_d.at[0])
    idx_load(1, 1, sem_i)
    pltpu.async_copy(h_hbm.at[idx_s.at[0, 0]], buf_a, sem_a)

    def sb_body(s, carry):
        p = lax.rem(s, 2)
        pn = lax.rem(s + 1, 2)
        for j in range(SB):
            if j == SB - 2:
                idx_wait(jnp.minimum(s + 1, NSB - 1), pn, sem_i)
            pltpu.make_async_copy(
                h_hbm.at[idx_s.at[p, j]], bufs[j % 2], sems[j % 2]).wait()
            if j < SB - 1:
                nxt_idx = idx_s.at[p, j + 1]
            else:
                nxt_idx = idx_s.at[pn, 0]
            pltpu.async_copy(h_hbm.at[nxt_idx], bufs[(j + 1) % 2],
                             sems[(j + 1) % 2])
            pltpu.sync_copy(bufs[j % 2], acc_sh.at[idx_d.at[p, j]],
                            add=True)
            deg_update(p, j)
        idx_load(jnp.minimum(s + 2, NSB - 1), p, sem_i)
        return carry

    lax.fori_loop(0, NSB, sb_body, 0)
    idx_wait(NSB - 1, (NSB - 1) % 2, sem_i)
    pltpu.make_async_copy(h_hbm.at[idx_s.at[0, 0]], buf_a, sem_a).wait()

    plsc.subcore_barrier()

    pltpu.sync_copy(acc_sh.at[pl.ds(sid * RPS, RPS)],
                    agg_out.at[cid, pl.ds(sid * RPS, RPS)])
    if compute_deg:
        pltpu.sync_copy(deg_v, deg_out.at[w])


def _make_sc_agg(compute_deg):
    out_type = [jax.ShapeDtypeStruct((NC, N_ACC, D), jnp.float32)]
    scratch = [
        pltpu.VMEM((2, SB, CHUNK), jnp.int32),
        pltpu.VMEM((2, SB, CHUNK), jnp.int32),
        pltpu.VMEM((CHUNK, D), jnp.float32),
        pltpu.VMEM((CHUNK, D), jnp.float32),
    ]
    if compute_deg:
        out_type.append(jax.ShapeDtypeStruct((NW, N_ACC), jnp.float32))
        scratch.append(pltpu.VMEM((N_ACC,), jnp.float32))
    scratch.append(pltpu.VMEM_SHARED((N_ACC, D), jnp.float32))
    scratch.append(pltpu.SemaphoreType.DMA)
    scratch.append(pltpu.SemaphoreType.DMA)
    scratch.append(pltpu.SemaphoreType.DMA)

    return pl.kernel(
        functools.partial(_sc_agg_body, compute_deg),
        out_type=out_type,
        mesh=plsc.VectorSubcoreMesh(core_axis_name="c", subcore_axis_name="s"),
        scratch_types=scratch,
        compiler_params=pltpu.CompilerParams(needs_layout_passes=False),
    )


_sc_agg_deg = _make_sc_agg(True)
_sc_agg = _make_sc_agg(False)


_R = 400


def _dense_body(h_ref, a_ref, d_ref, ws_ref, wn_ref, b_ref, o_ref):
    a = a_ref[0] + a_ref[1]
    deg = jnp.sum(d_ref[...], axis=1)
    inv = 1.0 / jnp.maximum(deg, 1.0)
    hn = a * inv[:, None]
    acc = jnp.dot(h_ref[...], ws_ref[...], preferred_element_type=jnp.float32)
    acc = acc + jnp.dot(hn, wn_ref[...], preferred_element_type=jnp.float32)
    o_ref[...] = jnp.maximum(acc + b_ref[0:1, :], 0.0)


def _dense(h, agg, deg_t, W_self, W_neigh, b2d):
    return pl.pallas_call(
        _dense_body,
        grid=(N // _R,),
        in_specs=[
            pl.BlockSpec((_R, D), lambda i: (i, 0)),
            pl.BlockSpec((NC, _R, D), lambda i: (0, i, 0)),
            pl.BlockSpec((_R, NW), lambda i: (i, 0)),
            pl.BlockSpec((D, D), lambda i: (0, 0)),
            pl.BlockSpec((D, D), lambda i: (0, 0)),
            pl.BlockSpec((8, D), lambda i: (0, 0)),
        ],
        out_specs=pl.BlockSpec((_R, D), lambda i: (i, 0)),
        out_shape=jax.ShapeDtypeStruct((N, D), jnp.float32),
    )(h, agg, deg_t, W_self, W_neigh, b2d)


@jax.jit
def kernel(g, inputs, W_self1, W_neigh1, b1, W_self2, W_neigh2, b2):
    pad = E_PAD - E
    pad_src = (jnp.arange(pad, dtype=jnp.int32) * 131) % N
    pad_dst = DUMMY + (jnp.arange(pad, dtype=jnp.int32) % 32)
    g_p = jnp.concatenate(
        [g, jnp.stack([pad_src, pad_dst])], axis=1).reshape(
        2, NW, NSB, SB, CHUNK)
    z = jnp.zeros((CHUNK, D), jnp.float32)
    b1_2 = jnp.broadcast_to(b1, (8, D))
    b2_2 = jnp.broadcast_to(b2, (8, D))

    agg1, deg_parts = _sc_agg_deg(g_p, inputs, z)
    deg_t = deg_parts.T
    h1 = _dense(inputs, agg1, deg_t, W_self1, W_neigh1, b1_2)
    (agg2,) = _sc_agg(g_p, h1, z)
    h2 = _dense(h1, agg2, deg_t, W_self2, W_neigh2, b2_2)
    return h2


# speedup vs baseline: 1.0456x; 1.0155x over previous
import functools

import jax
import jax.numpy as jnp
from jax import lax
from jax.experimental import pallas as pl
from jax.experimental.pallas import tpu as pltpu
from jax.experimental.pallas import tpu_sc as plsc

N = 10000
E = 320000
D = 128
NC = 2
NS = 16
NW = NC * NS
CHUNK = 128
CPW = 80
EPW = CPW * CHUNK
E_PAD = EPW * NW
N_ACC = 10240
RPS = N_ACC // NS
DUMMY = N + 8


SB = 8
NSB = CPW // SB


def _sc_agg_body(compute_deg, g_hbm, h_hbm, z_hbm, zd_hbm, *rest):
    if compute_deg:
        (agg_out, deg_out, idx_s, idx_d, buf_a, buf_b, deg_v, acc_sh,
         sem_a, sem_b, sem_i) = rest
    else:
        agg_out, idx_s, idx_d, buf_a, buf_b, acc_sh, sem_a, sem_b, sem_i = rest
        deg_out = deg_v = None

    cid = lax.axis_index("c")
    sid = lax.axis_index("s")
    w = cid * NS + sid
    bufs = (buf_a, buf_b)
    sems = (sem_a, sem_b)

    ones16 = jnp.ones((16,), jnp.float32)

    def deg_update(p, j):
        if compute_deg:
            for k in range(CHUNK // 16):
                idx16 = idx_d[p, j, pl.ds(k * 16, 16)]
                plsc.addupdate_scatter(deg_v, [idx16], ones16)

    def idx_load(s_clamped, half, sem):
        pltpu.async_copy(g_hbm.at[0, w, s_clamped], idx_s.at[half], sem)
        pltpu.async_copy(g_hbm.at[1, w, s_clamped], idx_d.at[half], sem)

    def idx_wait(s_clamped, half, sem):
        pltpu.make_async_copy(g_hbm.at[0, w, s_clamped], idx_s.at[half],
                              sem).wait()
        pltpu.make_async_copy(g_hbm.at[1, w, s_clamped], idx_d.at[half],
                              sem).wait()

    idx_load(0, 0, sem_b)
    idx_load(1, 1, sem_i)
    if compute_deg:
        pltpu.async_copy(zd_hbm, deg_v, sem_a)
    pltpu.sync_copy(z_hbm, buf_a)
    for r in range(RPS // CHUNK):
        pltpu.sync_copy(buf_a, acc_sh.at[pl.ds(sid * RPS + r * CHUNK, CHUNK)])
    if compute_deg:
        pltpu.make_async_copy(zd_hbm, deg_v, sem_a).wait()
    idx_wait(0, 0, sem_b)

    plsc.subcore_barrier()

    pltpu.async_copy(h_hbm.at[idx_s.at[0, 0]], buf_a, sem_a)

    def sb_body(s, carry):
        p = lax.rem(s, 2)
        pn = lax.rem(s + 1, 2)
        for j in range(SB):
            if j == SB - 2:
                idx_wait(jnp.minimum(s + 1, NSB - 1), pn, sem_i)
            pltpu.make_async_copy(
                h_hbm.at[idx_s.at[p, j]], bufs[j % 2], sems[j % 2]).wait()
            if j < SB - 1:
                nxt_idx = idx_s.at[p, j + 1]
            else:
                nxt_idx = idx_s.at[pn, 0]
            pltpu.async_copy(h_hbm.at[nxt_idx], bufs[(j + 1) % 2],
                             sems[(j + 1) % 2])
            pltpu.sync_copy(bufs[j % 2], acc_sh.at[idx_d.at[p, j]],
                            add=True)
            deg_update(p, j)
        idx_load(jnp.minimum(s + 2, NSB - 1), p, sem_i)
        return carry

    lax.fori_loop(0, NSB, sb_body, 0)
    idx_wait(NSB - 1, (NSB - 1) % 2, sem_i)
    pltpu.make_async_copy(h_hbm.at[idx_s.at[0, 0]], buf_a, sem_a).wait()

    plsc.subcore_barrier()

    pltpu.sync_copy(acc_sh.at[pl.ds(sid * RPS, RPS)],
                    agg_out.at[cid, pl.ds(sid * RPS, RPS)])
    if compute_deg:
        pltpu.sync_copy(deg_v, deg_out.at[w])


def _make_sc_agg(compute_deg):
    out_type = [jax.ShapeDtypeStruct((NC, N_ACC, D), jnp.float32)]
    scratch = [
        pltpu.VMEM((2, SB, CHUNK), jnp.int32),
        pltpu.VMEM((2, SB, CHUNK), jnp.int32),
        pltpu.VMEM((CHUNK, D), jnp.float32),
        pltpu.VMEM((CHUNK, D), jnp.float32),
    ]
    if compute_deg:
        out_type.append(jax.ShapeDtypeStruct((NW, N_ACC), jnp.float32))
        scratch.append(pltpu.VMEM((N_ACC,), jnp.float32))
    scratch.append(pltpu.VMEM_SHARED((N_ACC, D), jnp.float32))
    scratch.append(pltpu.SemaphoreType.DMA)
    scratch.append(pltpu.SemaphoreType.DMA)
    scratch.append(pltpu.SemaphoreType.DMA)

    return pl.kernel(
        functools.partial(_sc_agg_body, compute_deg),
        out_type=out_type,
        mesh=plsc.VectorSubcoreMesh(core_axis_name="c", subcore_axis_name="s"),
        scratch_types=scratch,
        compiler_params=pltpu.CompilerParams(needs_layout_passes=False),
    )


_sc_agg_deg = _make_sc_agg(True)
_sc_agg = _make_sc_agg(False)


_R = 400


def _dense_body(h_ref, a_ref, d_ref, ws_ref, wn_ref, b_ref, o_ref):
    a = a_ref[0] + a_ref[1]
    deg = jnp.sum(d_ref[...], axis=1)
    inv = 1.0 / jnp.maximum(deg, 1.0)
    hn = a * inv[:, None]
    acc = jnp.dot(h_ref[...], ws_ref[...], preferred_element_type=jnp.float32)
    acc = acc + jnp.dot(hn, wn_ref[...], preferred_element_type=jnp.float32)
    o_ref[...] = jnp.maximum(acc + b_ref[0:1, :], 0.0)


def _dense(h, agg, deg_t, W_self, W_neigh, b2d):
    return pl.pallas_call(
        _dense_body,
        grid=(N // _R,),
        in_specs=[
            pl.BlockSpec((_R, D), lambda i: (i, 0)),
            pl.BlockSpec((NC, _R, D), lambda i: (0, i, 0)),
            pl.BlockSpec((_R, NW), lambda i: (i, 0)),
            pl.BlockSpec((D, D), lambda i: (0, 0)),
            pl.BlockSpec((D, D), lambda i: (0, 0)),
            pl.BlockSpec((8, D), lambda i: (0, 0)),
        ],
        out_specs=pl.BlockSpec((_R, D), lambda i: (i, 0)),
        out_shape=jax.ShapeDtypeStruct((N, D), jnp.float32),
    )(h, agg, deg_t, W_self, W_neigh, b2d)


@jax.jit
def kernel(g, inputs, W_self1, W_neigh1, b1, W_self2, W_neigh2, b2):
    pad = E_PAD - E
    pad_src = (jnp.arange(pad, dtype=jnp.int32) * 131) % N
    pad_dst = DUMMY + (jnp.arange(pad, dtype=jnp.int32) % 32)
    g_p = jnp.concatenate(
        [g, jnp.stack([pad_src, pad_dst])], axis=1).reshape(
        2, NW, NSB, SB, CHUNK)
    z = jnp.zeros((CHUNK, D), jnp.float32)
    zd = jnp.zeros((N_ACC,), jnp.float32)
    b1_2 = jnp.broadcast_to(b1, (8, D))
    b2_2 = jnp.broadcast_to(b2, (8, D))

    agg1, deg_parts = _sc_agg_deg(g_p, inputs, z, zd)
    deg_t = deg_parts.T
    h1 = _dense(inputs, agg1, deg_t, W_self1, W_neigh1, b1_2)
    (agg2,) = _sc_agg(g_p, h1, z, zd)
    h2 = _dense(h1, agg2, deg_t, W_self2, W_neigh2, b2_2)
    return h2
